# Initial kernel scaffold; baseline (speedup 1.0000x reference)
import jax
import jax.numpy as jnp
from jax.experimental import pallas as pl


def kernel(source, H, reflection_id_grid):
    raise NotImplementedError("write your pallas kernel here")


# speedup vs baseline: 6.6606x; 6.6606x over previous
import jax
import jax.numpy as jnp
from jax import lax
from jax.experimental import pallas as pl
from jax.experimental.pallas import tpu as pltpu
from jax.experimental.pallas import tpu_sc as plsc

N_REFLN = 2_000_000
D = 16
GRID_DIM = 161
NC = 2
NS = 16
NW = NC * NS
B = 128
NBATCH = N_REFLN // B
FULL_ROUNDS = NBATCH // NW
REM_BATCHES = NBATCH - FULL_ROUNDS * NW


def _sc_body(h_hbm, grid_hbm, src_hbm, out_hbm,
             h_v, lin_v, gid_v, rows_v, sem_g, sem_r):
    c = lax.axis_index("c")
    s = lax.axis_index("s")
    wid = s * NC + c

    def do_batch(m):
        pltpu.sync_copy(h_hbm.at[pl.ds(m * (3 * B), 3 * B)], h_v)
        for j in range(B // 16):
            h0 = h_v[pl.ds(j * 16, 16)]
            h1 = h_v[pl.ds(B + j * 16, 16)]
            h2 = h_v[pl.ds(2 * B + j * 16, 16)]
            lin_v[pl.ds(j * 16, 16)] = (
                h0 * (GRID_DIM * GRID_DIM) + h1 * GRID_DIM + h2
            )
        pltpu.async_copy(grid_hbm.at[lin_v], gid_v, sem_g).wait()
        pltpu.async_copy(src_hbm.at[gid_v], rows_v, sem_r).wait()
        pltpu.sync_copy(rows_v, out_hbm.at[pl.ds(m * B, B)])

    def loop_body(i, carry):
        do_batch(i * NW + wid)
        return carry

    lax.fori_loop(0, FULL_ROUNDS, loop_body, 0)

    @pl.when(wid < REM_BATCHES)
    def _():
        do_batch(FULL_ROUNDS * NW + wid)


def kernel(source, H, reflection_id_grid):
    h_blk = (
        H.astype(jnp.int32)
        .reshape(NBATCH, B, 3)
        .transpose(0, 2, 1)
        .reshape(-1)
    )
    grid_flat = reflection_id_grid.reshape(-1)
    mesh = plsc.VectorSubcoreMesh(core_axis_name="c", subcore_axis_name="s")
    run = pl.kernel(
        _sc_body,
        mesh=mesh,
        compiler_params=pltpu.CompilerParams(use_tc_tiling_on_sc=False),
        out_type=jax.ShapeDtypeStruct((N_REFLN, D), jnp.float32),
        scratch_types=[
            pltpu.VMEM((3 * B,), jnp.int32),
            pltpu.VMEM((B,), jnp.int32),
            pltpu.VMEM((B,), jnp.int32),
            pltpu.VMEM((B, D), jnp.float32),
            pltpu.SemaphoreType.DMA,
            pltpu.SemaphoreType.DMA,
        ],
    )
    return run(h_blk, grid_flat, source)


# speedup vs baseline: 9.9355x; 1.4917x over previous
import jax
import jax.numpy as jnp
from jax import lax
from jax.experimental import pallas as pl
from jax.experimental.pallas import tpu as pltpu
from jax.experimental.pallas import tpu_sc as plsc

N_REFLN = 2_000_000
D = 16
GRID_DIM = 161
NC = 2
NS = 16
NW = NC * NS
B = 128
K = 4
SB = K * B
NBATCH = N_REFLN // B
PER_W = NBATCH // NW
REM = NBATCH - PER_W * NW
NSB = PER_W // K
PAIRS = NSB // 2


def _sc_body(h_hbm, grid_hbm, src_hbm, out_hbm,
             h_v, lin_v, gid_v, rows_v,
             sem_g0, sem_g1, sem_s0, sem_s1, sem_o0, sem_o1):
    c = lax.axis_index("c")
    s = lax.axis_index("s")
    wid = s * NC + c
    b0 = wid * PER_W

    hv = (h_v.at[0], h_v.at[1])
    lv = (lin_v.at[0], lin_v.at[1])
    gv = (gid_v.at[0], gid_v.at[1])
    rv = (rows_v.at[0], rows_v.at[1])
    sg = (sem_g0, sem_g1)
    ss = (sem_s0, sem_s1)
    so = (sem_o0, sem_o1)

    def load_and_lin(sb, p):
        pltpu.sync_copy(h_hbm.at[pl.ds(sb * (3 * SB), 3 * SB)], hv[p])
        for k in range(K):
            for j in range(B // 16):
                h0 = hv[p][pl.ds(k * 3 * B + j * 16, 16)]
                h1 = hv[p][pl.ds(k * 3 * B + B + j * 16, 16)]
                h2 = hv[p][pl.ds(k * 3 * B + 2 * B + j * 16, 16)]
                lv[p][pl.ds(k * B + j * 16, 16)] = (
                    h0 * (GRID_DIM * GRID_DIM) + h1 * GRID_DIM + h2
                )

    def fire_grid(p):
        return [
            pltpu.async_copy(
                grid_hbm.at[lv[p].at[pl.ds(k * B, B)]],
                gv[p].at[pl.ds(k * B, B)],
                sg[p],
            )
            for k in range(K)
        ]

    def fire_src(p):
        return [
            pltpu.async_copy(
                src_hbm.at[gv[p].at[pl.ds(k * B, B)]],
                rv[p].at[pl.ds(k * B, B)],
                ss[p],
            )
            for k in range(K)
        ]

    def fire_out(sb, p):
        return pltpu.async_copy(rv[p], out_hbm.at[pl.ds(sb * SB, SB)], so[p])

    def drain(copies):
        for cp in copies:
            cp.wait()

    def pair_body(t, carry):
        sb_a = b0 // K + 2 * t
        sb_b = sb_a + 1
        load_and_lin(sb_a, 0)
        ga = fire_grid(0)
        load_and_lin(sb_b, 1)
        drain(ga)
        sa = fire_src(0)
        gb = fire_grid(1)
        drain(sa)
        oa = fire_out(sb_a, 0)
        drain(gb)
        sb_ = fire_src(1)
        drain(sb_)
        ob = fire_out(sb_b, 1)
        oa.wait()
        ob.wait()
        return carry

    lax.fori_loop(0, PAIRS, pair_body, 0)

    @pl.when(wid < REM)
    def _():
        m = NBATCH - REM + wid
        pltpu.sync_copy(
            h_hbm.at[pl.ds(m * (3 * B), 3 * B)], hv[0].at[pl.ds(0, 3 * B)]
        )
        for j in range(B // 16):
            h0 = hv[0][pl.ds(j * 16, 16)]
            h1 = hv[0][pl.ds(B + j * 16, 16)]
            h2 = hv[0][pl.ds(2 * B + j * 16, 16)]
            lv[0][pl.ds(j * 16, 16)] = (
                h0 * (GRID_DIM * GRID_DIM) + h1 * GRID_DIM + h2
            )
        pltpu.async_copy(
            grid_hbm.at[lv[0].at[pl.ds(0, B)]], gv[0].at[pl.ds(0, B)], sg[0]
        ).wait()
        pltpu.async_copy(
            src_hbm.at[gv[0].at[pl.ds(0, B)]], rv[0].at[pl.ds(0, B)], ss[0]
        ).wait()
        pltpu.sync_copy(rv[0].at[pl.ds(0, B)], out_hbm.at[pl.ds(m * B, B)])


def kernel(source, H, reflection_id_grid):
    h_blk = (
        H.astype(jnp.int32)
        .reshape(NBATCH, B, 3)
        .transpose(0, 2, 1)
        .reshape(-1)
    )
    grid_flat = reflection_id_grid.reshape(-1)
    mesh = plsc.VectorSubcoreMesh(core_axis_name="c", subcore_axis_name="s")
    run = pl.kernel(
        _sc_body,
        mesh=mesh,
        compiler_params=pltpu.CompilerParams(use_tc_tiling_on_sc=False),
        out_type=jax.ShapeDtypeStruct((N_REFLN, D), jnp.float32),
        scratch_types=[
            pltpu.VMEM((2, 3 * SB), jnp.int32),
            pltpu.VMEM((2, SB), jnp.int32),
            pltpu.VMEM((2, SB), jnp.int32),
            pltpu.VMEM((2, SB, D), jnp.float32),
            pltpu.SemaphoreType.DMA,
            pltpu.SemaphoreType.DMA,
            pltpu.SemaphoreType.DMA,
            pltpu.SemaphoreType.DMA,
            pltpu.SemaphoreType.DMA,
            pltpu.SemaphoreType.DMA,
        ],
    )
    return run(h_blk, grid_flat, source)
